# baseline (speedup 1.0000x reference)
import functools

import jax
import jax.numpy as jnp
from jax import lax
from jax.experimental import pallas as pl
from jax.experimental.pallas import tpu as pltpu
from jax.experimental.pallas import tpu_sc as plsc

E = 64
K = 6
H = 128
M = 80
MS = 160
N = 2048
P = N * K
T = 64
NB = 256
PADN = NB * T
NW = 32
CPW = P // NW
TPW = N // NW


def _silu(x):
    return x * jax.nn.sigmoid(x)


def _gate_body(x_ref, wg_ref, ws1_ref, ws2_ref, ws3_ref,
               shared_ref, wp_ref, dstp_ref, be_ref):
    x = x_ref[...]
    xb = x.astype(jnp.bfloat16)
    g = jnp.dot(xb, ws1_ref[...].astype(jnp.bfloat16),
                preferred_element_type=jnp.float32)
    u = jnp.dot(xb, ws2_ref[...].astype(jnp.bfloat16),
                preferred_element_type=jnp.float32)
    shared_ref[...] = jnp.dot(
        (_silu(g) * u).astype(jnp.bfloat16),
        ws3_ref[...].astype(jnp.bfloat16),
        preferred_element_type=jnp.float32)

    logits = jax.lax.dot_general(
        x, wg_ref[...], (((1,), (1,)), ((), ())),
        preferred_element_type=jnp.float32)
    mx = jnp.max(logits, axis=-1, keepdims=True)
    p = jnp.exp(logits - mx)
    scores = p / jnp.sum(p, axis=-1, keepdims=True)

    lane = jax.lax.broadcasted_iota(jnp.int32, (N, E), 1)
    work = scores
    total = jnp.zeros((N, 1), jnp.float32)
    picks = []
    vals = []
    for _ in range(K):
        mval = jnp.max(work, axis=-1, keepdims=True)
        ismax = work == mval
        first = jnp.min(jnp.where(ismax, lane, E), axis=-1, keepdims=True)
        pick = lane == first
        picks.append(pick)
        vals.append(mval)
        total = total + mval
        work = jnp.where(pick, -1.0, work)
    sel = jnp.where(work < 0.0, 1.0, 0.0)
    inv_total = 1.0 / (total + 1e-20)
    lanes16 = jnp.ones((1, 16), jnp.float32)
    wp_ref[...] = jnp.concatenate(
        [(v * inv_total) * lanes16 for v in vals], axis=1)

    ones_row = jnp.ones((8, N), jnp.float32)
    c_e = jax.lax.dot_general(
        ones_row, sel, (((1,), (0,)), ((), ())),
        preferred_element_type=jnp.float32)[0:1]
    pad_e = jnp.floor((c_e + (T - 1)) * (1.0 / T)) * T
    lane_e = jax.lax.broadcasted_iota(jnp.int32, (E, E), 0)
    lane_f = jax.lax.broadcasted_iota(jnp.int32, (E, E), 1)
    upper = jnp.where(lane_e < lane_f, 1.0, 0.0)
    start = jax.lax.dot_general(
        pad_e, upper, (((1,), (0,)), ((), ())),
        preferred_element_type=jnp.float32)
    cum_incl = start + pad_e

    r = jax.lax.broadcasted_iota(jnp.int32, (T, T), 0)
    c = jax.lax.broadcasted_iota(jnp.int32, (T, T), 1)
    tril = jnp.where(r > c, 1.0, 0.0)
    blocks = []
    carry = jnp.zeros((1, E), jnp.float32)
    for b in range(N // T):
        sblk = sel[b * T:(b + 1) * T, :]
        rblk = jax.lax.dot_general(
            tril, sblk, (((1,), (0,)), ((), ())),
            preferred_element_type=jnp.float32) + carry
        blocks.append(rblk)
        carry = carry + jax.lax.dot_general(
            jnp.ones((1, T), jnp.float32), sblk, (((1,), (0,)), ((), ())),
            preferred_element_type=jnp.float32)
    rank = jnp.concatenate(blocks, axis=0)
    dst_mat = start + rank

    dcols = []
    for k in range(K):
        dcols.append(jnp.sum(jnp.where(picks[k], dst_mat, 0.0), axis=-1,
                             keepdims=True))
    dstp_ref[...] = jnp.concatenate(dcols, axis=1).astype(jnp.int32)

    bidx = (jax.lax.broadcasted_iota(jnp.int32, (8, 128), 0) * 128
            + jax.lax.broadcasted_iota(jnp.int32, (8, 128), 1))
    bpos = bidx.astype(jnp.float32) * float(T)
    be = jnp.zeros((8, 128), jnp.int32)
    for e in range(E):
        be = be + jnp.where(bpos >= cum_incl[0, e], 1, 0)
    be_ref[...] = jnp.minimum(be, E - 1)


def _gate(x, Wg, Ws1, Ws2, Ws3):
    return pl.pallas_call(
        _gate_body,
        out_shape=[
            jax.ShapeDtypeStruct((N, H), jnp.float32),
            jax.ShapeDtypeStruct((N, K * 16), jnp.float32),
            jax.ShapeDtypeStruct((N, K), jnp.int32),
            jax.ShapeDtypeStruct((8, 128), jnp.int32),
        ],
    )(x, Wg, Ws1, Ws2, Ws3)


def _dispatch_body(x_hbm, tok_hbm, dst_hbm, xs_hbm,
                   tok_v, dst_v, rows_v, gsem, ssem):
    wid = lax.axis_index("s") * 2 + lax.axis_index("c")
    c1 = pltpu.async_copy(tok_hbm.at[wid], tok_v, ssem)
    c2 = pltpu.async_copy(dst_hbm.at[wid], dst_v, ssem)
    c1.wait()
    c2.wait()
    nj = CPW // 128
    gathers = [pltpu.async_copy(x_hbm.at[tok_v.at[j]],
                                rows_v.at[pl.ds(j * 128, 128)], gsem)
               for j in range(nj)]
    scatters = []
    for j in range(nj):
        gathers[j].wait()
        scatters.append(
            pltpu.async_copy(rows_v.at[pl.ds(j * 128, 128)],
                             xs_hbm.at[dst_v.at[j]], ssem))
    for s in scatters:
        s.wait()


def _dispatch(x, tok3, dst3):
    mesh = plsc.VectorSubcoreMesh(core_axis_name="c", subcore_axis_name="s")
    f = functools.partial(
        pl.kernel, mesh=mesh,
        out_type=jax.ShapeDtypeStruct((PADN, H), jnp.float32),
        scratch_types=[
            pltpu.VMEM((CPW // 128, 128), jnp.int32),
            pltpu.VMEM((CPW // 128, 128), jnp.int32),
            pltpu.VMEM((CPW, H), jnp.float32),
            pltpu.SemaphoreType.DMA,
            pltpu.SemaphoreType.DMA,
        ],
    )(_dispatch_body)
    return f(x, tok3, dst3)


BPG = 32


def _gemm_body(be_ref, xs_ref, w1_ref, w2_ref, w3_ref, ys_ref):
    i = pl.program_id(0)
    for sub in range(BPG):
        e = be_ref[i * BPG + sub]
        xb = xs_ref[sub * T:(sub + 1) * T, :].astype(jnp.bfloat16)
        g = jnp.dot(xb, w1_ref[e], preferred_element_type=jnp.float32)
        u = jnp.dot(xb, w2_ref[e], preferred_element_type=jnp.float32)
        a = (_silu(g) * u).astype(jnp.bfloat16)
        ys_ref[sub * T:(sub + 1) * T, :] = jnp.dot(
            a, w3_ref[e], preferred_element_type=jnp.float32)


def _grouped_gemm(be, xs, W1b, W2b, W3b):
    grid_spec = pltpu.PrefetchScalarGridSpec(
        num_scalar_prefetch=1,
        grid=(NB // BPG,),
        in_specs=[
            pl.BlockSpec((BPG * T, H), lambda i, be: (i, 0)),
            pl.BlockSpec((E, H, M), lambda i, be: (0, 0, 0)),
            pl.BlockSpec((E, H, M), lambda i, be: (0, 0, 0)),
            pl.BlockSpec((E, M, H), lambda i, be: (0, 0, 0)),
        ],
        out_specs=pl.BlockSpec((BPG * T, H), lambda i, be: (i, 0)),
    )
    return pl.pallas_call(
        _gemm_body,
        grid_spec=grid_spec,
        out_shape=jax.ShapeDtypeStruct((PADN, H), jnp.float32),
        compiler_params=pltpu.CompilerParams(
            dimension_semantics=("arbitrary",)),
    )(be, xs, W1b, W2b, W3b)


def _combine_body(ys_hbm, dst_hbm, w_hbm, sh_hbm, y_hbm,
                  dst_v, w_v, rows_v, out_v, sem):
    wid = lax.axis_index("s") * 2 + lax.axis_index("c")
    c1 = pltpu.async_copy(dst_hbm.at[wid], dst_v, sem)
    c2 = pltpu.async_copy(w_hbm.at[wid], w_v, sem)
    c3 = pltpu.async_copy(sh_hbm.at[pl.ds(wid * TPW, TPW)], out_v, sem)
    c1.wait()
    c2.wait()
    c3.wait()
    gathers = [pltpu.async_copy(ys_hbm.at[dst_v.at[j]],
                                rows_v.at[pl.ds(j * 128, 128)], sem)
               for j in range(CPW // 128)]
    for g in gathers:
        g.wait()

    def tok_body(t, _):
        vs = [out_v[t, pl.ds(16 * j, 16)] for j in range(H // 16)]
        for k in range(K):
            p = t * K + k
            wv = w_v[p, :]
            for j in range(H // 16):
                vs[j] = vs[j] + wv * rows_v[p, pl.ds(16 * j, 16)]
        for j in range(H // 16):
            out_v[t, pl.ds(16 * j, 16)] = vs[j]
        return 0

    lax.fori_loop(0, TPW, tok_body, 0)
    pltpu.sync_copy(out_v, y_hbm.at[pl.ds(wid * TPW, TPW)])


def _combine(ys, dst3, wflat, shared):
    mesh = plsc.VectorSubcoreMesh(core_axis_name="c", subcore_axis_name="s")
    f = functools.partial(
        pl.kernel, mesh=mesh,
        out_type=jax.ShapeDtypeStruct((N, H), jnp.float32),
        scratch_types=[
            pltpu.VMEM((CPW // 128, 128), jnp.int32),
            pltpu.VMEM((CPW, 16), jnp.float32),
            pltpu.VMEM((CPW, H), jnp.float32),
            pltpu.VMEM((TPW, H), jnp.float32),
            pltpu.SemaphoreType.DMA,
        ],
    )(_combine_body)
    return f(ys, dst3, wflat, shared)


def kernel(hidden_states, Wg, W1, W2, W3, Ws1, Ws2, Ws3):
    B, S, h = hidden_states.shape
    x = hidden_states.reshape(N, H)
    shared, wp, dstp, be8 = _gate(x, Wg, Ws1, Ws2, Ws3)
    tok3 = (jnp.arange(P, dtype=jnp.int32) // K).reshape(NW, CPW // 128, 128)
    dst3 = dstp.reshape(NW, CPW // 128, 128)
    wrows = wp.reshape(NW, CPW, 16)
    be = be8.reshape(1024)
    xs = _dispatch(x, tok3, dst3)
    ys = _grouped_gemm(be, xs, W1.astype(jnp.bfloat16),
                       W2.astype(jnp.bfloat16), W3.astype(jnp.bfloat16))
    y = _combine(ys, dst3, wrows, shared)
    return y.reshape(B, S, h)


# speedup vs baseline: 1.1130x; 1.1130x over previous
import functools

import jax
import jax.numpy as jnp
from jax import lax
from jax.experimental import pallas as pl
from jax.experimental.pallas import tpu as pltpu
from jax.experimental.pallas import tpu_sc as plsc

E = 64
K = 6
H = 128
M = 80
MS = 160
N = 2048
P = N * K
T = 128
NB = 160
PADN = NB * T
NW = 32
CPW = P // NW
TPW = N // NW


def _silu(x):
    return x * jax.nn.sigmoid(x)


def _gate_body(x_ref, wg_ref, ws1_ref, ws2_ref, ws3_ref,
               shared_ref, wp_ref, dstp_ref, be_ref):
    x = x_ref[...]
    xb = x.astype(jnp.bfloat16)
    g = jnp.dot(xb, ws1_ref[...].astype(jnp.bfloat16),
                preferred_element_type=jnp.float32)
    u = jnp.dot(xb, ws2_ref[...].astype(jnp.bfloat16),
                preferred_element_type=jnp.float32)
    shared_ref[...] = jnp.dot(
        (_silu(g) * u).astype(jnp.bfloat16),
        ws3_ref[...].astype(jnp.bfloat16),
        preferred_element_type=jnp.float32)

    logits = jax.lax.dot_general(
        x, wg_ref[...], (((1,), (1,)), ((), ())),
        preferred_element_type=jnp.float32)
    mx = jnp.max(logits, axis=-1, keepdims=True)
    p = jnp.exp(logits - mx)
    scores = p / jnp.sum(p, axis=-1, keepdims=True)

    lane = jax.lax.broadcasted_iota(jnp.int32, (N, E), 1)
    work = scores
    total = jnp.zeros((N, 1), jnp.float32)
    picks = []
    vals = []
    for _ in range(K):
        mval = jnp.max(work, axis=-1, keepdims=True)
        ismax = work == mval
        first = jnp.min(jnp.where(ismax, lane, E), axis=-1, keepdims=True)
        pick = lane == first
        picks.append(pick)
        vals.append(mval)
        total = total + mval
        work = jnp.where(pick, -1.0, work)
    sel = jnp.where(work < 0.0, 1.0, 0.0)
    inv_total = 1.0 / (total + 1e-20)
    lanes16 = jnp.ones((1, 16), jnp.float32)
    wp_ref[...] = jnp.concatenate(
        [(v * inv_total) * lanes16 for v in vals], axis=1)

    ones_row = jnp.ones((8, N), jnp.float32)
    c_e = jax.lax.dot_general(
        ones_row, sel, (((1,), (0,)), ((), ())),
        preferred_element_type=jnp.float32)[0:1]
    pad_e = jnp.floor((c_e + (T - 1)) * (1.0 / T)) * T
    lane_e = jax.lax.broadcasted_iota(jnp.int32, (E, E), 0)
    lane_f = jax.lax.broadcasted_iota(jnp.int32, (E, E), 1)
    upper = jnp.where(lane_e < lane_f, 1.0, 0.0)
    start = jax.lax.dot_general(
        pad_e, upper, (((1,), (0,)), ((), ())),
        preferred_element_type=jnp.float32)
    cum_incl = start + pad_e

    r = jax.lax.broadcasted_iota(jnp.int32, (T, T), 0)
    c = jax.lax.broadcasted_iota(jnp.int32, (T, T), 1)
    tril = jnp.where(r > c, 1.0, 0.0)
    blocks = []
    carry = jnp.zeros((1, E), jnp.float32)
    for b in range(N // T):
        sblk = sel[b * T:(b + 1) * T, :]
        rblk = jax.lax.dot_general(
            tril, sblk, (((1,), (0,)), ((), ())),
            preferred_element_type=jnp.float32) + carry
        blocks.append(rblk)
        carry = carry + jax.lax.dot_general(
            jnp.ones((1, T), jnp.float32), sblk, (((1,), (0,)), ((), ())),
            preferred_element_type=jnp.float32)
    rank = jnp.concatenate(blocks, axis=0)
    dst_mat = start + rank

    dcols = []
    for k in range(K):
        dcols.append(jnp.sum(jnp.where(picks[k], dst_mat, 0.0), axis=-1,
                             keepdims=True))
    dstp_ref[...] = jnp.concatenate(dcols, axis=1).astype(jnp.int32)

    bidx = (jax.lax.broadcasted_iota(jnp.int32, (8, 128), 0) * 128
            + jax.lax.broadcasted_iota(jnp.int32, (8, 128), 1))
    bpos = bidx.astype(jnp.float32) * float(T)
    be = jnp.zeros((8, 128), jnp.int32)
    for e in range(E):
        be = be + jnp.where(bpos >= cum_incl[0, e], 1, 0)
    be_ref[...] = jnp.minimum(be, E - 1)


def _gate(x, Wg, Ws1, Ws2, Ws3):
    return pl.pallas_call(
        _gate_body,
        out_shape=[
            jax.ShapeDtypeStruct((N, H), jnp.float32),
            jax.ShapeDtypeStruct((N, K * 16), jnp.float32),
            jax.ShapeDtypeStruct((N, K), jnp.int32),
            jax.ShapeDtypeStruct((8, 128), jnp.int32),
        ],
    )(x, Wg, Ws1, Ws2, Ws3)


def _dispatch_body(x_hbm, tok_hbm, dst_hbm, xs_hbm,
                   tok_v, dst_v, rows_v, gsem, ssem):
    wid = lax.axis_index("s") * 2 + lax.axis_index("c")
    c1 = pltpu.async_copy(tok_hbm.at[wid], tok_v, ssem)
    c2 = pltpu.async_copy(dst_hbm.at[wid], dst_v, ssem)
    c1.wait()
    c2.wait()
    nj = CPW // 128
    gathers = [pltpu.async_copy(x_hbm.at[tok_v.at[j]],
                                rows_v.at[pl.ds(j * 128, 128)], gsem)
               for j in range(nj)]
    scatters = []
    for j in range(nj):
        gathers[j].wait()
        scatters.append(
            pltpu.async_copy(rows_v.at[pl.ds(j * 128, 128)],
                             xs_hbm.at[dst_v.at[j]], ssem))
    for s in scatters:
        s.wait()


def _dispatch(x, tok3, dst3):
    mesh = plsc.VectorSubcoreMesh(core_axis_name="c", subcore_axis_name="s")
    f = functools.partial(
        pl.kernel, mesh=mesh,
        out_type=jax.ShapeDtypeStruct((PADN, H), jnp.float32),
        scratch_types=[
            pltpu.VMEM((CPW // 128, 128), jnp.int32),
            pltpu.VMEM((CPW // 128, 128), jnp.int32),
            pltpu.VMEM((CPW, H), jnp.float32),
            pltpu.SemaphoreType.DMA,
            pltpu.SemaphoreType.DMA,
        ],
    )(_dispatch_body)
    return f(x, tok3, dst3)


BPG = 16


def _gemm_body(be_ref, xs_ref, w1_ref, w2_ref, w3_ref, ys_ref):
    i = pl.program_id(0)
    for sub in range(BPG):
        e = be_ref[i * BPG + sub]
        xb = xs_ref[sub * T:(sub + 1) * T, :].astype(jnp.bfloat16)
        g = jnp.dot(xb, w1_ref[e], preferred_element_type=jnp.float32)
        u = jnp.dot(xb, w2_ref[e], preferred_element_type=jnp.float32)
        a = (_silu(g) * u).astype(jnp.bfloat16)
        ys_ref[sub * T:(sub + 1) * T, :] = jnp.dot(
            a, w3_ref[e], preferred_element_type=jnp.float32)


def _grouped_gemm(be, xs, W1b, W2b, W3b):
    grid_spec = pltpu.PrefetchScalarGridSpec(
        num_scalar_prefetch=1,
        grid=(NB // BPG,),
        in_specs=[
            pl.BlockSpec((BPG * T, H), lambda i, be: (i, 0)),
            pl.BlockSpec((E, H, M), lambda i, be: (0, 0, 0)),
            pl.BlockSpec((E, H, M), lambda i, be: (0, 0, 0)),
            pl.BlockSpec((E, M, H), lambda i, be: (0, 0, 0)),
        ],
        out_specs=pl.BlockSpec((BPG * T, H), lambda i, be: (i, 0)),
    )
    return pl.pallas_call(
        _gemm_body,
        grid_spec=grid_spec,
        out_shape=jax.ShapeDtypeStruct((PADN, H), jnp.float32),
        compiler_params=pltpu.CompilerParams(
            dimension_semantics=("arbitrary",)),
    )(be, xs, W1b, W2b, W3b)


def _combine_body(ys_hbm, dst_hbm, w_hbm, sh_hbm, y_hbm,
                  dst_v, w_v, rows_v, out_v, sem):
    wid = lax.axis_index("s") * 2 + lax.axis_index("c")
    c1 = pltpu.async_copy(dst_hbm.at[wid], dst_v, sem)
    c2 = pltpu.async_copy(w_hbm.at[wid], w_v, sem)
    c3 = pltpu.async_copy(sh_hbm.at[pl.ds(wid * TPW, TPW)], out_v, sem)
    c1.wait()
    c2.wait()
    c3.wait()
    gathers = [pltpu.async_copy(ys_hbm.at[dst_v.at[j]],
                                rows_v.at[pl.ds(j * 128, 128)], sem)
               for j in range(CPW // 128)]
    for g in gathers:
        g.wait()

    def tok_body(t, _):
        vs = [out_v[t, pl.ds(16 * j, 16)] for j in range(H // 16)]
        for k in range(K):
            p = t * K + k
            wv = w_v[p, :]
            for j in range(H // 16):
                vs[j] = vs[j] + wv * rows_v[p, pl.ds(16 * j, 16)]
        for j in range(H // 16):
            out_v[t, pl.ds(16 * j, 16)] = vs[j]
        return 0

    lax.fori_loop(0, TPW, tok_body, 0)
    pltpu.sync_copy(out_v, y_hbm.at[pl.ds(wid * TPW, TPW)])


def _combine(ys, dst3, wflat, shared):
    mesh = plsc.VectorSubcoreMesh(core_axis_name="c", subcore_axis_name="s")
    f = functools.partial(
        pl.kernel, mesh=mesh,
        out_type=jax.ShapeDtypeStruct((N, H), jnp.float32),
        scratch_types=[
            pltpu.VMEM((CPW // 128, 128), jnp.int32),
            pltpu.VMEM((CPW, 16), jnp.float32),
            pltpu.VMEM((CPW, H), jnp.float32),
            pltpu.VMEM((TPW, H), jnp.float32),
            pltpu.SemaphoreType.DMA,
        ],
    )(_combine_body)
    return f(ys, dst3, wflat, shared)


def kernel(hidden_states, Wg, W1, W2, W3, Ws1, Ws2, Ws3):
    B, S, h = hidden_states.shape
    x = hidden_states.reshape(N, H)
    shared, wp, dstp, be8 = _gate(x, Wg, Ws1, Ws2, Ws3)
    tok3 = (jnp.arange(P, dtype=jnp.int32) // K).reshape(NW, CPW // 128, 128)
    dst3 = dstp.reshape(NW, CPW // 128, 128)
    wrows = wp.reshape(NW, CPW, 16)
    be = be8.reshape(1024)
    xs = _dispatch(x, tok3, dst3)
    ys = _grouped_gemm(be, xs, W1.astype(jnp.bfloat16),
                       W2.astype(jnp.bfloat16), W3.astype(jnp.bfloat16))
    y = _combine(ys, dst3, wrows, shared)
    return y.reshape(B, S, h)


# speedup vs baseline: 1.2363x; 1.1108x over previous
import functools

import jax
import jax.numpy as jnp
from jax import lax
from jax.experimental import pallas as pl
from jax.experimental.pallas import tpu as pltpu
from jax.experimental.pallas import tpu_sc as plsc

E = 64
K = 6
H = 128
M = 80
MS = 160
N = 2048
P = N * K
T = 128
NB = 160
PADN = NB * T
NW = 32
CPW = P // NW
TPW = N // NW


def _silu(x):
    return x * jax.nn.sigmoid(x)


def _gate_body(x_ref, wg_ref, ws1_ref, ws2_ref, ws3_ref,
               shared_ref, wp_ref, dstp_ref, be_ref):
    x = x_ref[...]
    xb = x.astype(jnp.bfloat16)
    g = jnp.dot(xb, ws1_ref[...].astype(jnp.bfloat16),
                preferred_element_type=jnp.float32)
    u = jnp.dot(xb, ws2_ref[...].astype(jnp.bfloat16),
                preferred_element_type=jnp.float32)
    shared_ref[...] = jnp.dot(
        (_silu(g) * u).astype(jnp.bfloat16),
        ws3_ref[...].astype(jnp.bfloat16),
        preferred_element_type=jnp.float32)

    logits = jax.lax.dot_general(
        x, wg_ref[...], (((1,), (1,)), ((), ())),
        preferred_element_type=jnp.float32)
    mx = jnp.max(logits, axis=-1, keepdims=True)
    p = jnp.exp(logits - mx)
    scores = p / jnp.sum(p, axis=-1, keepdims=True)

    lane = jax.lax.broadcasted_iota(jnp.int32, (N, E), 1)
    work = scores
    total = jnp.zeros((N, 1), jnp.float32)
    picks = []
    vals = []
    for _ in range(K):
        mval = jnp.max(work, axis=-1, keepdims=True)
        ismax = work == mval
        first = jnp.min(jnp.where(ismax, lane, E), axis=-1, keepdims=True)
        pick = lane == first
        picks.append(pick)
        vals.append(mval)
        total = total + mval
        work = jnp.where(pick, -1.0, work)
    sel = jnp.where(work < 0.0, 1.0, 0.0)
    inv_total = 1.0 / (total + 1e-20)
    lanes16 = jnp.ones((1, 16), jnp.float32)
    wp_ref[...] = jnp.concatenate(
        [(v * inv_total) * lanes16 for v in vals], axis=1)

    ones_row = jnp.ones((8, N), jnp.float32)
    c_e = jax.lax.dot_general(
        ones_row, sel, (((1,), (0,)), ((), ())),
        preferred_element_type=jnp.float32)[0:1]
    pad_e = jnp.floor((c_e + (T - 1)) * (1.0 / T)) * T
    lane_e = jax.lax.broadcasted_iota(jnp.int32, (E, E), 0)
    lane_f = jax.lax.broadcasted_iota(jnp.int32, (E, E), 1)
    upper = jnp.where(lane_e < lane_f, 1.0, 0.0)
    start = jax.lax.dot_general(
        pad_e, upper, (((1,), (0,)), ((), ())),
        preferred_element_type=jnp.float32)
    cum_incl = start + pad_e

    r = jax.lax.broadcasted_iota(jnp.int32, (T, T), 0)
    c = jax.lax.broadcasted_iota(jnp.int32, (T, T), 1)
    tril = jnp.where(r > c, 1.0, 0.0)
    blocks = []
    carry = jnp.zeros((1, E), jnp.float32)
    for b in range(N // T):
        sblk = sel[b * T:(b + 1) * T, :]
        rblk = jax.lax.dot_general(
            tril, sblk, (((1,), (0,)), ((), ())),
            preferred_element_type=jnp.float32) + carry
        blocks.append(rblk)
        carry = carry + jax.lax.dot_general(
            jnp.ones((1, T), jnp.float32), sblk, (((1,), (0,)), ((), ())),
            preferred_element_type=jnp.float32)
    rank = jnp.concatenate(blocks, axis=0)
    dst_mat = start + rank

    dcols = []
    for k in range(K):
        dcols.append(jnp.sum(jnp.where(picks[k], dst_mat, 0.0), axis=-1,
                             keepdims=True))
    dstp_ref[...] = jnp.concatenate(dcols, axis=1).astype(jnp.int32)

    bidx = (jax.lax.broadcasted_iota(jnp.int32, (8, 128), 0) * 128
            + jax.lax.broadcasted_iota(jnp.int32, (8, 128), 1))
    bpos = bidx.astype(jnp.float32) * float(T)
    be = jnp.zeros((8, 128), jnp.int32)
    for e in range(E):
        be = be + jnp.where(bpos >= cum_incl[0, e], 1, 0)
    nb_used = (cum_incl[0, E - 1] * (1.0 / T)).astype(jnp.int32)
    be_ref[...] = jnp.where(bidx == 1023, nb_used, jnp.minimum(be, E - 1))


def _gate(x, Wg, Ws1, Ws2, Ws3):
    return pl.pallas_call(
        _gate_body,
        out_shape=[
            jax.ShapeDtypeStruct((N, H), jnp.float32),
            jax.ShapeDtypeStruct((N, K * 16), jnp.float32),
            jax.ShapeDtypeStruct((N, K), jnp.int32),
            jax.ShapeDtypeStruct((8, 128), jnp.int32),
        ],
    )(x, Wg, Ws1, Ws2, Ws3)


def _dispatch_body(x_hbm, tok_hbm, dst_hbm, xs_hbm,
                   tok_v, dst_v, rows_v, gsem, ssem):
    wid = lax.axis_index("s") * 2 + lax.axis_index("c")
    c1 = pltpu.async_copy(tok_hbm.at[wid], tok_v, ssem)
    c2 = pltpu.async_copy(dst_hbm.at[wid], dst_v, ssem)
    c1.wait()
    c2.wait()
    nj = CPW // 128
    gathers = [pltpu.async_copy(x_hbm.at[tok_v.at[j]],
                                rows_v.at[pl.ds(j * 128, 128)], gsem)
               for j in range(nj)]
    scatters = []
    for j in range(nj):
        gathers[j].wait()
        scatters.append(
            pltpu.async_copy(rows_v.at[pl.ds(j * 128, 128)],
                             xs_hbm.at[dst_v.at[j]], ssem))
    for s in scatters:
        s.wait()


def _dispatch(x, tok3, dst3):
    mesh = plsc.VectorSubcoreMesh(core_axis_name="c", subcore_axis_name="s")
    f = functools.partial(
        pl.kernel, mesh=mesh,
        out_type=jax.ShapeDtypeStruct((PADN, H), jnp.float32),
        scratch_types=[
            pltpu.VMEM((CPW // 128, 128), jnp.int32),
            pltpu.VMEM((CPW // 128, 128), jnp.int32),
            pltpu.VMEM((CPW, H), jnp.float32),
            pltpu.SemaphoreType.DMA,
            pltpu.SemaphoreType.DMA,
        ],
    )(_dispatch_body)
    return f(x, tok3, dst3)


BPG = 8


def _gemm_body(be_ref, xs_ref, w1_ref, w2_ref, w3_ref, ys_ref):
    i = pl.program_id(0)

    @pl.when(i * BPG < be_ref[1023])
    def _():
        xall = xs_ref[...].astype(jnp.bfloat16)
        gs, us, es = [], [], []
        for sub in range(BPG):
            e = be_ref[i * BPG + sub]
            xb = xall[sub * T:(sub + 1) * T, :]
            gs.append(jnp.dot(xb, w1_ref[e],
                              preferred_element_type=jnp.float32))
            us.append(jnp.dot(xb, w2_ref[e],
                              preferred_element_type=jnp.float32))
            es.append(e)
        for sub in range(BPG):
            a = (_silu(gs[sub]) * us[sub]).astype(jnp.bfloat16)
            ys_ref[sub * T:(sub + 1) * T, :] = jnp.dot(
                a, w3_ref[es[sub]], preferred_element_type=jnp.float32)


def _grouped_gemm(be, xs, W1b, W2b, W3b):
    grid_spec = pltpu.PrefetchScalarGridSpec(
        num_scalar_prefetch=1,
        grid=(NB // BPG,),
        in_specs=[
            pl.BlockSpec((BPG * T, H), lambda i, be: (i, 0)),
            pl.BlockSpec((E, H, M), lambda i, be: (0, 0, 0)),
            pl.BlockSpec((E, H, M), lambda i, be: (0, 0, 0)),
            pl.BlockSpec((E, M, H), lambda i, be: (0, 0, 0)),
        ],
        out_specs=pl.BlockSpec((BPG * T, H), lambda i, be: (i, 0)),
    )
    return pl.pallas_call(
        _gemm_body,
        grid_spec=grid_spec,
        out_shape=jax.ShapeDtypeStruct((PADN, H), jnp.float32),
        compiler_params=pltpu.CompilerParams(
            dimension_semantics=("arbitrary",)),
    )(be, xs, W1b, W2b, W3b)


def _combine_body(ys_hbm, dst_hbm, w_hbm, sh_hbm, y_hbm,
                  dst_v, w_v, rows_v, out_v, sem):
    wid = lax.axis_index("s") * 2 + lax.axis_index("c")
    c1 = pltpu.async_copy(dst_hbm.at[wid], dst_v, sem)
    c2 = pltpu.async_copy(w_hbm.at[wid], w_v, sem)
    c3 = pltpu.async_copy(sh_hbm.at[pl.ds(wid * TPW, TPW)], out_v, sem)
    c1.wait()
    c2.wait()
    c3.wait()
    gathers = [pltpu.async_copy(ys_hbm.at[dst_v.at[j]],
                                rows_v.at[pl.ds(j * 128, 128)], sem)
               for j in range(CPW // 128)]
    for g in gathers:
        g.wait()

    def tok_body(t, _):
        vs = [out_v[t, pl.ds(16 * j, 16)] for j in range(H // 16)]
        for k in range(K):
            p = t * K + k
            wv = w_v[p, :]
            for j in range(H // 16):
                vs[j] = vs[j] + wv * rows_v[p, pl.ds(16 * j, 16)]
        for j in range(H // 16):
            out_v[t, pl.ds(16 * j, 16)] = vs[j]
        return 0

    lax.fori_loop(0, TPW, tok_body, 0)
    pltpu.sync_copy(out_v, y_hbm.at[pl.ds(wid * TPW, TPW)])


def _combine(ys, dst3, wflat, shared):
    mesh = plsc.VectorSubcoreMesh(core_axis_name="c", subcore_axis_name="s")
    f = functools.partial(
        pl.kernel, mesh=mesh,
        out_type=jax.ShapeDtypeStruct((N, H), jnp.float32),
        scratch_types=[
            pltpu.VMEM((CPW // 128, 128), jnp.int32),
            pltpu.VMEM((CPW, 16), jnp.float32),
            pltpu.VMEM((CPW, H), jnp.float32),
            pltpu.VMEM((TPW, H), jnp.float32),
            pltpu.SemaphoreType.DMA,
        ],
    )(_combine_body)
    return f(ys, dst3, wflat, shared)


def kernel(hidden_states, Wg, W1, W2, W3, Ws1, Ws2, Ws3):
    B, S, h = hidden_states.shape
    x = hidden_states.reshape(N, H)
    shared, wp, dstp, be8 = _gate(x, Wg, Ws1, Ws2, Ws3)
    tok3 = (jnp.arange(P, dtype=jnp.int32) // K).reshape(NW, CPW // 128, 128)
    dst3 = dstp.reshape(NW, CPW // 128, 128)
    wrows = wp.reshape(NW, CPW, 16)
    be = be8.reshape(1024)
    xs = _dispatch(x, tok3, dst3)
    ys = _grouped_gemm(be, xs, W1.astype(jnp.bfloat16),
                       W2.astype(jnp.bfloat16), W3.astype(jnp.bfloat16))
    y = _combine(ys, dst3, wrows, shared)
    return y.reshape(B, S, h)
